# Initial kernel scaffold; baseline (speedup 1.0000x reference)
import jax
import jax.numpy as jnp
from jax.experimental import pallas as pl


def kernel(x, edge_index, batch, W1, b1, W2, b2):
    raise NotImplementedError("write your pallas kernel here")


# speedup vs baseline: 9.1420x; 9.1420x over previous
import functools

import jax
import jax.numpy as jnp
from jax import lax
from jax.experimental import pallas as pl
from jax.experimental.pallas import tpu as pltpu
from jax.experimental.pallas import tpu_sc as plsc

N = 10000
D = 128
E = 320000
G = 64
K = 30
NP = 10240
CE = 125
ER = E // CE
RPT = ER // 32
ROWS_PER_TILE = NP // 16
ZR = 128
CP = 80
OUTROWS = 1984
TRASH = G * K

_mesh = lambda: plsc.VectorSubcoreMesh(core_axis_name="c", subcore_axis_name="s")


def _deg_kernel(dst2):
    @functools.partial(
        pl.kernel,
        mesh=_mesh(),
        compiler_params=pltpu.CompilerParams(use_tc_tiling_on_sc=False),
        out_type=jax.ShapeDtypeStruct((2, NP, 16), jnp.float32),
        scratch_types=[
            pltpu.VMEM((RPT, CE), jnp.int32),
            pltpu.VMEM((CE, 16), jnp.float32),
            pltpu.VMEM((ZR, 16), jnp.float32),
            pltpu.VMEM_SHARED((NP, 16), jnp.float32),
        ],
    )
    def k(dst_hbm, out_hbm, didx, ones, zbuf, acc):
        c = lax.axis_index("c")
        s = lax.axis_index("s")
        zeros16 = jnp.zeros((16,), jnp.float32)
        ones16 = jnp.ones((16,), jnp.float32)

        def fill(i, _):
            ones[i, :] = ones16
            return _
        lax.fori_loop(0, CE, fill, None)

        def zfill(i, _):
            zbuf[i, :] = zeros16
            return _
        lax.fori_loop(0, ZR, zfill, None)

        def zacc(q, _):
            pltpu.sync_copy(zbuf, acc.at[pl.ds(s * ROWS_PER_TILE + q * ZR, ZR)])
            return _
        lax.fori_loop(0, ROWS_PER_TILE // ZR, zacc, None)
        plsc.subcore_barrier()

        base = (c * 16 + s) * RPT
        pltpu.sync_copy(dst_hbm.at[pl.ds(base, RPT)], didx)

        def body(j, _):
            pltpu.sync_copy(ones, acc.at[didx.at[j]], add=True)
            return _
        lax.fori_loop(0, RPT, body, None)

        plsc.subcore_barrier()
        sl = pl.ds(s * ROWS_PER_TILE, ROWS_PER_TILE)
        pltpu.sync_copy(acc.at[sl], out_hbm.at[c, sl])

    return k(dst2)


DH = D // 2
RPT2 = ER // 16


def _agg_kernel(yr, sidx2, dst2):
    @functools.partial(
        pl.kernel,
        mesh=_mesh(),
        compiler_params=pltpu.CompilerParams(use_tc_tiling_on_sc=False),
        out_type=jax.ShapeDtypeStruct((2, NP, DH), jnp.float32),
        scratch_types=[
            pltpu.VMEM((RPT2, CE), jnp.int32),
            pltpu.VMEM((RPT2, CE), jnp.int32),
            pltpu.VMEM((CE, DH), jnp.float32),
            pltpu.VMEM((ZR, DH), jnp.float32),
            pltpu.VMEM_SHARED((NP, DH), jnp.float32),
            pltpu.SemaphoreType.DMA,
        ],
    )
    def k(y_hbm, src_hbm, dst_hbm, out_hbm, sidx, didx, rows, zbuf, acc, sem):
        c = lax.axis_index("c")
        s = lax.axis_index("s")
        zeros16 = jnp.zeros((16,), jnp.float32)

        def zrow(i, _):
            def zcol(j, __):
                zbuf[i, pl.ds(j * 16, 16)] = zeros16
                return __
            return lax.fori_loop(0, DH // 16, zcol, _)
        lax.fori_loop(0, ZR, zrow, None)

        def zacc(q, _):
            pltpu.sync_copy(zbuf, acc.at[pl.ds(s * ROWS_PER_TILE + q * ZR, ZR)])
            return _
        lax.fori_loop(0, ROWS_PER_TILE // ZR, zacc, None)
        plsc.subcore_barrier()

        base = s * RPT2
        pltpu.sync_copy(src_hbm.at[c, pl.ds(base, RPT2)], sidx)
        pltpu.sync_copy(dst_hbm.at[pl.ds(base, RPT2)], didx)

        def body(j, _):
            pltpu.async_copy(y_hbm.at[sidx.at[j]], rows, sem).wait()
            pltpu.sync_copy(rows, acc.at[didx.at[j]], add=True)
            return _
        lax.fori_loop(0, RPT2, body, None)

        plsc.subcore_barrier()
        sl = pl.ds(s * ROWS_PER_TILE, ROWS_PER_TILE)
        pltpu.sync_copy(acc.at[sl], out_hbm.at[c, sl])

    return k(yr, sidx2, dst2)


def _pool_scatter_kernel(h2, fo2):
    @functools.partial(
        pl.kernel,
        mesh=_mesh(),
        compiler_params=pltpu.CompilerParams(use_tc_tiling_on_sc=False),
        out_type=jax.ShapeDtypeStruct((OUTROWS, D), jnp.float32),
        scratch_types=[
            pltpu.VMEM((8, CP), jnp.int32),
            pltpu.VMEM((124, D), jnp.float32),
            pltpu.VMEM((CP, D), jnp.float32),
        ],
    )
    def k(h2_hbm, fo_hbm, out_hbm, fidx, zbuf, rows):
        c = lax.axis_index("c")
        s = lax.axis_index("s")
        zeros16 = jnp.zeros((16,), jnp.float32)

        @pl.when(c == 0)
        def _():
            def zrow(i, _):
                def zcol(j, __):
                    zbuf[i, pl.ds(j * 16, 16)] = zeros16
                    return __
                return lax.fori_loop(0, D // 16, zcol, _)
            lax.fori_loop(0, 124, zrow, None)
            pltpu.sync_copy(zbuf, out_hbm.at[pl.ds(s * 124, 124)])
            plsc.subcore_barrier()

            pltpu.sync_copy(fo_hbm.at[pl.ds(s * 8, 8)], fidx)

            def body(q, _):
                pltpu.sync_copy(h2_hbm.at[pl.ds(s * ROWS_PER_TILE + q * CP, CP)], rows)
                pltpu.sync_copy(rows, out_hbm.at[fidx.at[q]])
                return _
            lax.fori_loop(0, 8, body, None)

    return k(h2, fo2)


def _scale_matmul_kernel(hist, x_p, W1):
    def body(hist_ref, x_ref, w_ref, y_ref, dis_ref):
        h = hist_ref[...]
        deg = h[0, :, 0] + h[1, :, 0] + 1.0
        dis = lax.rsqrt(deg)
        y = jnp.dot(x_ref[...], w_ref[...], preferred_element_type=jnp.float32)
        y_ref[...] = y * dis[:, None]
        dis_ref[...] = dis[:, None]

    return pl.pallas_call(
        body,
        grid=(NP // 1024,),
        in_specs=[
            pl.BlockSpec((2, 1024, 16), lambda i: (0, i, 0)),
            pl.BlockSpec((1024, D), lambda i: (i, 0)),
            pl.BlockSpec((D, D), lambda i: (0, 0)),
        ],
        out_specs=[
            pl.BlockSpec((1024, D), lambda i: (i, 0)),
            pl.BlockSpec((1024, 1), lambda i: (i, 0)),
        ],
        out_shape=[
            jax.ShapeDtypeStruct((NP, D), jnp.float32),
            jax.ShapeDtypeStruct((NP, 1), jnp.float32),
        ],
    )(hist, x_p, W1)


def _mid_kernel(p, y1, dis, b1, W2):
    def body(p_ref, y1_ref, dis_ref, b1_ref, w_ref, y2_ref):
        ph = p_ref[...]
        agg = jnp.concatenate([ph[0], ph[1]], axis=1)
        dis = dis_ref[...]
        h1 = dis * (agg + y1_ref[...]) + b1_ref[...]
        h1 = jnp.maximum(h1, 0.0)
        y2 = jnp.dot(h1, w_ref[...], preferred_element_type=jnp.float32)
        y2_ref[...] = y2 * dis

    blk = lambda i: (i, 0)
    return pl.pallas_call(
        body,
        grid=(NP // 1024,),
        in_specs=[
            pl.BlockSpec((2, 1024, DH), lambda i: (0, i, 0)),
            pl.BlockSpec((1024, D), blk),
            pl.BlockSpec((1024, 1), blk),
            pl.BlockSpec((1, D), lambda i: (0, 0)),
            pl.BlockSpec((D, D), lambda i: (0, 0)),
        ],
        out_specs=pl.BlockSpec((1024, D), blk),
        out_shape=jax.ShapeDtypeStruct((NP, D), jnp.float32),
    )(p, y1, dis, b1, W2)


def _final_kernel(q, y2, dis, b2):
    def body(q_ref, y2_ref, dis_ref, b2_ref, h2_ref, v_ref):
        qh = q_ref[...]
        agg = jnp.concatenate([qh[0], qh[1]], axis=1)
        h2 = dis_ref[...] * (agg + y2_ref[...]) + b2_ref[...]
        h2_ref[...] = h2
        v_ref[...] = h2[:, D - 1:D]

    blk = lambda i: (i, 0)
    return pl.pallas_call(
        body,
        grid=(NP // 1024,),
        in_specs=[
            pl.BlockSpec((2, 1024, DH), lambda i: (0, i, 0)),
            pl.BlockSpec((1024, D), blk),
            pl.BlockSpec((1024, 1), blk),
            pl.BlockSpec((1, D), lambda i: (0, 0)),
        ],
        out_specs=[
            pl.BlockSpec((1024, D), blk),
            pl.BlockSpec((1024, 1), blk),
        ],
        out_shape=[
            jax.ShapeDtypeStruct((NP, D), jnp.float32),
            jax.ShapeDtypeStruct((NP, 1), jnp.float32),
        ],
    )(q, y2, dis, b2)


def _rank_kernel(vrow, brow, vcol, bcol):
    NJ = NP // 2048

    def body(vr_ref, br_ref, vc_ref, bc_ref, fo_ref, acc_ref):
        i = pl.program_id(0)
        j = pl.program_id(1)
        vi = vc_ref[...]
        bi = bc_ref[...]
        vj = vr_ref[...]
        bj = br_ref[...]
        jg = j * 2048 + lax.broadcasted_iota(jnp.int32, (1, 2048), 1)
        ig = i * 256 + lax.broadcasted_iota(jnp.int32, (256, 1), 0)
        same = bj == bi
        before = (vj > vi) | ((vj == vi) & (jg < ig))
        cnt = jnp.sum((same & before).astype(jnp.int32), axis=1, keepdims=True)

        @pl.when(j == 0)
        def _():
            acc_ref[...] = cnt

        @pl.when(j > 0)
        def _():
            acc_ref[...] = acc_ref[...] + cnt

        @pl.when(j == NJ - 1)
        def _():
            rank = acc_ref[...]
            ok = (rank < K) & (bi < G)
            fo_ref[...] = jnp.where(ok, bi * K + rank, TRASH)

    return pl.pallas_call(
        body,
        grid=(NP // 256, NJ),
        in_specs=[
            pl.BlockSpec((1, 2048), lambda i, j: (0, j)),
            pl.BlockSpec((1, 2048), lambda i, j: (0, j)),
            pl.BlockSpec((256, 1), lambda i, j: (i, 0)),
            pl.BlockSpec((256, 1), lambda i, j: (i, 0)),
        ],
        out_specs=pl.BlockSpec((256, 1), lambda i, j: (i, 0)),
        out_shape=jax.ShapeDtypeStruct((NP, 1), jnp.int32),
        scratch_shapes=[pltpu.VMEM((256, 1), jnp.int32)],
    )(vrow, brow, vcol, bcol)


def kernel(x, edge_index, batch, W1, b1, W2, b2):
    src = edge_index[0].astype(jnp.int32)
    dst = edge_index[1].astype(jnp.int32)
    si = src * 2
    sidx2 = jnp.stack([si, si + 1]).reshape(2, ER, CE)
    dst2 = dst.reshape(ER, CE)
    b32 = batch.astype(jnp.int32)
    bpad = jnp.concatenate([b32, jnp.full((NP - N,), G, jnp.int32)])
    x_p = jnp.concatenate([x, jnp.zeros((NP - N, D), x.dtype)])

    hist = _deg_kernel(dst2)
    y1, dis = _scale_matmul_kernel(hist, x_p, W1.astype(jnp.float32))

    p = _agg_kernel(y1.reshape(2 * NP, DH), sidx2, dst2)
    y2 = _mid_kernel(p, y1, dis, b1.reshape(1, D), W2.astype(jnp.float32))

    q = _agg_kernel(y2.reshape(2 * NP, DH), sidx2, dst2)
    h2, v = _final_kernel(q, y2, dis, b2.reshape(1, D))

    fo = _rank_kernel(v.reshape(1, NP), bpad.reshape(1, NP), v, bpad.reshape(NP, 1))
    out = _pool_scatter_kernel(h2, fo.reshape(NP // CP, CP))
    return out[:TRASH].reshape(G, K * D)


# speedup vs baseline: 18.5650x; 2.0307x over previous
import functools

import jax
import jax.numpy as jnp
from jax import lax
from jax.experimental import pallas as pl
from jax.experimental.pallas import tpu as pltpu
from jax.experimental.pallas import tpu_sc as plsc

N = 10000
D = 128
E = 320000
G = 64
K = 30
NP = 10240
CE = 125
ER = E // CE
RPT = ER // 32
ROWS_PER_TILE = NP // 16
ZR = 128
CP = 80
OUTROWS = 1984
TRASH = G * K

_mesh = lambda: plsc.VectorSubcoreMesh(core_axis_name="c", subcore_axis_name="s")


def _deg_kernel(dst2):
    @functools.partial(
        pl.kernel,
        mesh=_mesh(),
        compiler_params=pltpu.CompilerParams(use_tc_tiling_on_sc=False),
        out_type=jax.ShapeDtypeStruct((2, NP, 16), jnp.float32),
        scratch_types=[
            pltpu.VMEM((RPT, CE), jnp.int32),
            pltpu.VMEM((CE, 16), jnp.float32),
            pltpu.VMEM((ZR, 16), jnp.float32),
            pltpu.VMEM_SHARED((NP, 16), jnp.float32),
        ],
    )
    def k(dst_hbm, out_hbm, didx, ones, zbuf, acc):
        c = lax.axis_index("c")
        s = lax.axis_index("s")
        zeros16 = jnp.zeros((16,), jnp.float32)
        ones16 = jnp.ones((16,), jnp.float32)

        def fill(i, _):
            ones[i, :] = ones16
            return _
        lax.fori_loop(0, CE, fill, None)

        def zfill(i, _):
            zbuf[i, :] = zeros16
            return _
        lax.fori_loop(0, ZR, zfill, None)

        def zacc(q, _):
            pltpu.sync_copy(zbuf, acc.at[pl.ds(s * ROWS_PER_TILE + q * ZR, ZR)])
            return _
        lax.fori_loop(0, ROWS_PER_TILE // ZR, zacc, None)
        plsc.subcore_barrier()

        base = (c * 16 + s) * RPT
        pltpu.sync_copy(dst_hbm.at[pl.ds(base, RPT)], didx)

        def body(j, _):
            pltpu.sync_copy(ones, acc.at[didx.at[j]], add=True)
            return _
        lax.fori_loop(0, RPT, body, None)

        plsc.subcore_barrier()
        sl = pl.ds(s * ROWS_PER_TILE, ROWS_PER_TILE)
        pltpu.sync_copy(acc.at[sl], out_hbm.at[c, sl])

    return k(dst2)


DH = D // 2
RPT2 = ER // 16


def _agg_kernel(yr, sidx2, dst2):
    @functools.partial(
        pl.kernel,
        mesh=_mesh(),
        compiler_params=pltpu.CompilerParams(use_tc_tiling_on_sc=False),
        out_type=jax.ShapeDtypeStruct((2, NP, DH), jnp.float32),
        scratch_types=[
            pltpu.VMEM((RPT2, CE), jnp.int32),
            pltpu.VMEM((RPT2, CE), jnp.int32),
            pltpu.VMEM((CE, DH), jnp.float32),
            pltpu.VMEM((CE, DH), jnp.float32),
            pltpu.VMEM((ZR, DH), jnp.float32),
            pltpu.VMEM_SHARED((NP, DH), jnp.float32),
            pltpu.SemaphoreType.DMA,
            pltpu.SemaphoreType.DMA,
        ],
    )
    def k(y_hbm, src_hbm, dst_hbm, out_hbm, sidx, didx, rows_a, rows_b,
          zbuf, acc, sem_a, sem_b):
        c = lax.axis_index("c")
        s = lax.axis_index("s")
        zeros16 = jnp.zeros((16,), jnp.float32)

        def zrow(i, _):
            def zcol(j, __):
                zbuf[i, pl.ds(j * 16, 16)] = zeros16
                return __
            return lax.fori_loop(0, DH // 16, zcol, _)
        lax.fori_loop(0, ZR, zrow, None)

        def zacc(q, _):
            pltpu.sync_copy(zbuf, acc.at[pl.ds(s * ROWS_PER_TILE + q * ZR, ZR)])
            return _
        lax.fori_loop(0, ROWS_PER_TILE // ZR, zacc, None)
        plsc.subcore_barrier()

        base = s * RPT2
        pltpu.sync_copy(src_hbm.at[c, pl.ds(base, RPT2)], sidx)
        pltpu.sync_copy(dst_hbm.at[pl.ds(base, RPT2)], didx)

        pltpu.async_copy(y_hbm.at[sidx.at[0]], rows_a, sem_a)

        def body(t, _):
            ja = 2 * t
            jb = 2 * t + 1
            pltpu.async_copy(y_hbm.at[sidx.at[jb]], rows_b, sem_b)
            pltpu.make_async_copy(y_hbm.at[sidx.at[ja]], rows_a, sem_a).wait()
            pltpu.sync_copy(rows_a, acc.at[didx.at[ja]], add=True)
            jn = jnp.minimum(ja + 2, RPT2 - 1)
            pltpu.async_copy(y_hbm.at[sidx.at[jn]], rows_a, sem_a)
            pltpu.make_async_copy(y_hbm.at[sidx.at[jb]], rows_b, sem_b).wait()
            pltpu.sync_copy(rows_b, acc.at[didx.at[jb]], add=True)
            return _
        lax.fori_loop(0, RPT2 // 2, body, None)
        pltpu.make_async_copy(y_hbm.at[sidx.at[RPT2 - 1]], rows_a, sem_a).wait()

        plsc.subcore_barrier()
        sl = pl.ds(s * ROWS_PER_TILE, ROWS_PER_TILE)
        pltpu.sync_copy(acc.at[sl], out_hbm.at[c, sl])

    return k(yr, sidx2, dst2)


def _pool_scatter_kernel(h2, fo2):
    @functools.partial(
        pl.kernel,
        mesh=_mesh(),
        compiler_params=pltpu.CompilerParams(use_tc_tiling_on_sc=False),
        out_type=jax.ShapeDtypeStruct((OUTROWS, D), jnp.float32),
        scratch_types=[
            pltpu.VMEM((8, CP), jnp.int32),
            pltpu.VMEM((124, D), jnp.float32),
            pltpu.VMEM((CP, D), jnp.float32),
        ],
    )
    def k(h2_hbm, fo_hbm, out_hbm, fidx, zbuf, rows):
        c = lax.axis_index("c")
        s = lax.axis_index("s")
        zeros16 = jnp.zeros((16,), jnp.float32)

        @pl.when(c == 0)
        def _():
            def zrow(i, _):
                def zcol(j, __):
                    zbuf[i, pl.ds(j * 16, 16)] = zeros16
                    return __
                return lax.fori_loop(0, D // 16, zcol, _)
            lax.fori_loop(0, 124, zrow, None)
            pltpu.sync_copy(zbuf, out_hbm.at[pl.ds(s * 124, 124)])
            plsc.subcore_barrier()

            pltpu.sync_copy(fo_hbm.at[pl.ds(s * 8, 8)], fidx)

            def body(q, _):
                pltpu.sync_copy(h2_hbm.at[pl.ds(s * ROWS_PER_TILE + q * CP, CP)], rows)
                pltpu.sync_copy(rows, out_hbm.at[fidx.at[q]])
                return _
            lax.fori_loop(0, 8, body, None)

    return k(h2, fo2)


def _scale_matmul_kernel(hist, x_p, W1):
    def body(hist_ref, x_ref, w_ref, y_ref, dis_ref):
        h = hist_ref[...]
        deg = h[0, :, 0] + h[1, :, 0] + 1.0
        dis = lax.rsqrt(deg)
        y = jnp.dot(x_ref[...], w_ref[...], preferred_element_type=jnp.float32)
        y_ref[...] = y * dis[:, None]
        dis_ref[...] = dis[:, None]

    return pl.pallas_call(
        body,
        grid=(NP // 1024,),
        in_specs=[
            pl.BlockSpec((2, 1024, 16), lambda i: (0, i, 0)),
            pl.BlockSpec((1024, D), lambda i: (i, 0)),
            pl.BlockSpec((D, D), lambda i: (0, 0)),
        ],
        out_specs=[
            pl.BlockSpec((1024, D), lambda i: (i, 0)),
            pl.BlockSpec((1024, 1), lambda i: (i, 0)),
        ],
        out_shape=[
            jax.ShapeDtypeStruct((NP, D), jnp.float32),
            jax.ShapeDtypeStruct((NP, 1), jnp.float32),
        ],
    )(hist, x_p, W1)


def _mid_kernel(p, y1, dis, b1, W2):
    def body(p_ref, y1_ref, dis_ref, b1_ref, w_ref, y2_ref):
        ph = p_ref[...]
        agg = jnp.concatenate([ph[0], ph[1]], axis=1)
        dis = dis_ref[...]
        h1 = dis * (agg + y1_ref[...]) + b1_ref[...]
        h1 = jnp.maximum(h1, 0.0)
        y2 = jnp.dot(h1, w_ref[...], preferred_element_type=jnp.float32)
        y2_ref[...] = y2 * dis

    blk = lambda i: (i, 0)
    return pl.pallas_call(
        body,
        grid=(NP // 1024,),
        in_specs=[
            pl.BlockSpec((2, 1024, DH), lambda i: (0, i, 0)),
            pl.BlockSpec((1024, D), blk),
            pl.BlockSpec((1024, 1), blk),
            pl.BlockSpec((1, D), lambda i: (0, 0)),
            pl.BlockSpec((D, D), lambda i: (0, 0)),
        ],
        out_specs=pl.BlockSpec((1024, D), blk),
        out_shape=jax.ShapeDtypeStruct((NP, D), jnp.float32),
    )(p, y1, dis, b1, W2)


def _final_kernel(q, y2, dis, b2):
    def body(q_ref, y2_ref, dis_ref, b2_ref, h2_ref, v_ref):
        qh = q_ref[...]
        agg = jnp.concatenate([qh[0], qh[1]], axis=1)
        h2 = dis_ref[...] * (agg + y2_ref[...]) + b2_ref[...]
        h2_ref[...] = h2
        v_ref[...] = h2[:, D - 1:D]

    blk = lambda i: (i, 0)
    return pl.pallas_call(
        body,
        grid=(NP // 1024,),
        in_specs=[
            pl.BlockSpec((2, 1024, DH), lambda i: (0, i, 0)),
            pl.BlockSpec((1024, D), blk),
            pl.BlockSpec((1024, 1), blk),
            pl.BlockSpec((1, D), lambda i: (0, 0)),
        ],
        out_specs=[
            pl.BlockSpec((1024, D), blk),
            pl.BlockSpec((1024, 1), blk),
        ],
        out_shape=[
            jax.ShapeDtypeStruct((NP, D), jnp.float32),
            jax.ShapeDtypeStruct((NP, 1), jnp.float32),
        ],
    )(q, y2, dis, b2)


def _rank_kernel(vrow, brow, vcol, bcol):
    NJ = NP // 2048

    def body(vr_ref, br_ref, vc_ref, bc_ref, fo_ref, acc_ref):
        i = pl.program_id(0)
        j = pl.program_id(1)
        vi = vc_ref[...]
        bi = bc_ref[...]
        vj = vr_ref[...]
        bj = br_ref[...]
        @pl.when(j == 0)
        def _():
            acc_ref[...] = jnp.zeros((256, 1), jnp.int32)

        overlap = (bj[0, 2047] >= bi[0, 0]) & (bj[0, 0] <= bi[255, 0])

        @pl.when(overlap)
        def _():
            jg = j * 2048 + lax.broadcasted_iota(jnp.int32, (1, 2048), 1)
            ig = i * 256 + lax.broadcasted_iota(jnp.int32, (256, 1), 0)
            same = bj == bi
            before = (vj > vi) | ((vj == vi) & (jg < ig))
            cnt = jnp.sum((same & before).astype(jnp.int32), axis=1, keepdims=True)
            acc_ref[...] = acc_ref[...] + cnt

        @pl.when(j == NJ - 1)
        def _():
            rank = acc_ref[...]
            ok = (rank < K) & (bi < G)
            ig2 = i * 256 + lax.broadcasted_iota(jnp.int32, (256, 1), 0)
            trash = TRASH + ig2 // ROWS_PER_TILE
            fo_ref[...] = jnp.where(ok, bi * K + rank, trash)

    return pl.pallas_call(
        body,
        grid=(NP // 256, NJ),
        in_specs=[
            pl.BlockSpec((1, 2048), lambda i, j: (0, j)),
            pl.BlockSpec((1, 2048), lambda i, j: (0, j)),
            pl.BlockSpec((256, 1), lambda i, j: (i, 0)),
            pl.BlockSpec((256, 1), lambda i, j: (i, 0)),
        ],
        out_specs=pl.BlockSpec((256, 1), lambda i, j: (i, 0)),
        out_shape=jax.ShapeDtypeStruct((NP, 1), jnp.int32),
        scratch_shapes=[pltpu.VMEM((256, 1), jnp.int32)],
    )(vrow, brow, vcol, bcol)


def kernel(x, edge_index, batch, W1, b1, W2, b2):
    src = edge_index[0].astype(jnp.int32)
    dst = edge_index[1].astype(jnp.int32)
    si = src * 2
    sidx2 = jnp.stack([si, si + 1]).reshape(2, ER, CE)
    dst2 = dst.reshape(ER, CE)
    b32 = batch.astype(jnp.int32)
    bpad = jnp.concatenate([b32, jnp.full((NP - N,), G, jnp.int32)])
    x_p = jnp.concatenate([x, jnp.zeros((NP - N, D), x.dtype)])

    hist = _deg_kernel(dst2)
    y1, dis = _scale_matmul_kernel(hist, x_p, W1.astype(jnp.float32))

    p = _agg_kernel(y1.reshape(2 * NP, DH), sidx2, dst2)
    y2 = _mid_kernel(p, y1, dis, b1.reshape(1, D), W2.astype(jnp.float32))

    q = _agg_kernel(y2.reshape(2 * NP, DH), sidx2, dst2)
    h2, v = _final_kernel(q, y2, dis, b2.reshape(1, D))

    fo = _rank_kernel(v.reshape(1, NP), bpad.reshape(1, NP), v, bpad.reshape(NP, 1))
    out = _pool_scatter_kernel(h2, fo.reshape(NP // CP, CP))
    return out[:TRASH].reshape(G, K * D)


# speedup vs baseline: 20.4551x; 1.1018x over previous
import functools

import jax
import jax.numpy as jnp
from jax import lax
from jax.experimental import pallas as pl
from jax.experimental.pallas import tpu as pltpu
from jax.experimental.pallas import tpu_sc as plsc

N = 10000
D = 128
E = 320000
G = 64
K = 30
NP = 10240
CE = 125
ER = E // CE
RPT = ER // 32
ROWS_PER_TILE = NP // 16
ZR = 128
CP = 80
OUTROWS = 1984
TRASH = G * K

_mesh = lambda: plsc.VectorSubcoreMesh(core_axis_name="c", subcore_axis_name="s")


def _deg_kernel(dst2):
    @functools.partial(
        pl.kernel,
        mesh=_mesh(),
        compiler_params=pltpu.CompilerParams(use_tc_tiling_on_sc=False),
        out_type=jax.ShapeDtypeStruct((2, NP, 16), jnp.float32),
        scratch_types=[
            pltpu.VMEM((RPT, CE), jnp.int32),
            pltpu.VMEM((CE, 16), jnp.float32),
            pltpu.VMEM((ZR, 16), jnp.float32),
            pltpu.VMEM_SHARED((NP, 16), jnp.float32),
            pltpu.SemaphoreType.DMA,
        ],
    )
    def k(dst_hbm, out_hbm, didx, ones, zbuf, acc, sem):
        c = lax.axis_index("c")
        s = lax.axis_index("s")
        zeros16 = jnp.zeros((16,), jnp.float32)
        ones16 = jnp.ones((16,), jnp.float32)

        def fill(i, _):
            ones[i, :] = ones16
            return _
        lax.fori_loop(0, CE, fill, None)

        def zfill(i, _):
            zbuf[i, :] = zeros16
            return _
        lax.fori_loop(0, ZR, zfill, None)

        def zacc(q, _):
            pltpu.sync_copy(zbuf, acc.at[pl.ds(s * ROWS_PER_TILE + q * ZR, ZR)])
            return _
        lax.fori_loop(0, ROWS_PER_TILE // ZR, zacc, None)
        plsc.subcore_barrier()

        base = (c * 16 + s) * RPT
        pltpu.sync_copy(dst_hbm.at[pl.ds(base, RPT)], didx)

        def body(j, _):
            pltpu.async_copy(ones, acc.at[didx.at[j]], sem, add=True)
            return _
        lax.fori_loop(0, RPT, body, None)

        def drain(j, _):
            pltpu.make_async_copy(ones, acc.at[didx.at[j]], sem).wait()
            return _
        lax.fori_loop(0, RPT, drain, None)

        plsc.subcore_barrier()
        sl = pl.ds(s * ROWS_PER_TILE, ROWS_PER_TILE)
        pltpu.sync_copy(acc.at[sl], out_hbm.at[c, sl])

    return k(dst2)


DH = D // 2
RPT2 = ER // 16


def _agg_kernel(yr, sidx2, dst2):
    @functools.partial(
        pl.kernel,
        mesh=_mesh(),
        compiler_params=pltpu.CompilerParams(use_tc_tiling_on_sc=False),
        out_type=jax.ShapeDtypeStruct((2, NP, DH), jnp.float32),
        scratch_types=[
            pltpu.VMEM((RPT2, CE), jnp.int32),
            pltpu.VMEM((RPT2, CE), jnp.int32),
            pltpu.VMEM((CE, DH), jnp.float32),
            pltpu.VMEM((CE, DH), jnp.float32),
            pltpu.VMEM((CE, DH), jnp.float32),
            pltpu.VMEM((CE, DH), jnp.float32),
            pltpu.VMEM((ZR, DH), jnp.float32),
            pltpu.VMEM_SHARED((NP, DH), jnp.float32),
            pltpu.SemaphoreType.DMA,
            pltpu.SemaphoreType.DMA,
            pltpu.SemaphoreType.DMA,
            pltpu.SemaphoreType.DMA,
            pltpu.SemaphoreType.DMA,
            pltpu.SemaphoreType.DMA,
            pltpu.SemaphoreType.DMA,
            pltpu.SemaphoreType.DMA,
        ],
    )
    def k(y_hbm, src_hbm, dst_hbm, out_hbm, sidx, didx, r0, r1, r2, r3,
          zbuf, acc, g0, g1, g2, g3, s0, s1, s2, s3):
        c = lax.axis_index("c")
        s = lax.axis_index("s")
        zeros16 = jnp.zeros((16,), jnp.float32)

        def zrow(i, _):
            def zcol(j, __):
                zbuf[i, pl.ds(j * 16, 16)] = zeros16
                return __
            return lax.fori_loop(0, DH // 16, zcol, _)
        lax.fori_loop(0, ZR, zrow, None)

        def zacc(q, _):
            pltpu.sync_copy(zbuf, acc.at[pl.ds(s * ROWS_PER_TILE + q * ZR, ZR)])
            return _
        lax.fori_loop(0, ROWS_PER_TILE // ZR, zacc, None)
        plsc.subcore_barrier()

        base = s * RPT2
        pltpu.sync_copy(src_hbm.at[c, pl.ds(base, RPT2)], sidx)
        pltpu.sync_copy(dst_hbm.at[pl.ds(base, RPT2)], didx)

        bufs = (r0, r1, r2, r3)
        gsem = (g0, g1, g2, g3)
        ssem = (s0, s1, s2, s3)

        def gth(j, u):
            return pltpu.async_copy(y_hbm.at[sidx.at[j]], bufs[u], gsem[u])

        def gth_wait(j, u):
            pltpu.make_async_copy(y_hbm.at[sidx.at[j]], bufs[u], gsem[u]).wait()

        def sct(j, u):
            return pltpu.async_copy(bufs[u], acc.at[didx.at[j]], ssem[u],
                                    add=True)

        def sct_wait(j, u):
            pltpu.make_async_copy(bufs[u], acc.at[didx.at[j]], ssem[u]).wait()

        gth(0, 0)
        gth(1, 1)
        gth_wait(0, 0)
        sct(0, 0)
        gth(2, 2)
        gth_wait(1, 1)
        sct(1, 1)
        gth(3, 3)

        def body(t, _):
            for u_ in range(4):
                j = 4 * t + 2 + u_
                u = (2 + u_) % 4
                w = u_
                sct_wait(j - 2, w)
                gth(j + 2, w)
                gth_wait(j, u)
                sct(j, u)
            return _
        lax.fori_loop(0, (RPT2 - 4) // 4, body, None)

        gth_wait(RPT2 - 2, (RPT2 - 2) % 4)
        sct(RPT2 - 2, (RPT2 - 2) % 4)
        gth_wait(RPT2 - 1, (RPT2 - 1) % 4)
        sct(RPT2 - 1, (RPT2 - 1) % 4)
        for u_ in range(4):
            sct_wait(RPT2 - 4 + u_, (RPT2 - 4 + u_) % 4)

        plsc.subcore_barrier()
        sl = pl.ds(s * ROWS_PER_TILE, ROWS_PER_TILE)
        pltpu.sync_copy(acc.at[sl], out_hbm.at[c, sl])

    return k(yr, sidx2, dst2)


def _pool_scatter_kernel(h2, fo2):
    @functools.partial(
        pl.kernel,
        mesh=_mesh(),
        compiler_params=pltpu.CompilerParams(use_tc_tiling_on_sc=False),
        out_type=jax.ShapeDtypeStruct((OUTROWS, D), jnp.float32),
        scratch_types=[
            pltpu.VMEM((8, CP), jnp.int32),
            pltpu.VMEM((124, D), jnp.float32),
            pltpu.VMEM((CP, D), jnp.float32),
            pltpu.VMEM((CP, D), jnp.float32),
            pltpu.SemaphoreType.DMA,
            pltpu.SemaphoreType.DMA,
        ],
    )
    def k(h2_hbm, fo_hbm, out_hbm, fidx, zbuf, rows_a, rows_b, sem_a, sem_b):
        c = lax.axis_index("c")
        s = lax.axis_index("s")
        zeros16 = jnp.zeros((16,), jnp.float32)

        @pl.when(c == 0)
        def _():
            def zrow(i, _):
                def zcol(j, __):
                    zbuf[i, pl.ds(j * 16, 16)] = zeros16
                    return __
                return lax.fori_loop(0, D // 16, zcol, _)
            lax.fori_loop(0, 124, zrow, None)
            pltpu.sync_copy(fo_hbm.at[pl.ds(s * 8, 8)], fidx)
            pltpu.sync_copy(zbuf, out_hbm.at[pl.ds(s * 124, 124)])
            plsc.subcore_barrier()

            def src_at(q):
                return h2_hbm.at[pl.ds(s * ROWS_PER_TILE + q * CP, CP)]

            pltpu.async_copy(src_at(0), rows_a, sem_a)

            def body(t, _):
                qa = 2 * t
                qb = 2 * t + 1
                pltpu.async_copy(src_at(qb), rows_b, sem_b)
                pltpu.make_async_copy(src_at(qa), rows_a, sem_a).wait()
                pltpu.sync_copy(rows_a, out_hbm.at[fidx.at[qa]])
                qn = jnp.minimum(qa + 2, 7)
                pltpu.async_copy(src_at(qn), rows_a, sem_a)
                pltpu.make_async_copy(src_at(qb), rows_b, sem_b).wait()
                pltpu.sync_copy(rows_b, out_hbm.at[fidx.at[qb]])
                return _
            lax.fori_loop(0, 4, body, None)
            pltpu.make_async_copy(src_at(7), rows_a, sem_a).wait()

    return k(h2, fo2)


def _scale_matmul_kernel(hist, x_p, W1):
    def body(hist_ref, x_ref, w_ref, y_ref, dis_ref):
        h = hist_ref[...]
        deg = h[0, :, 0] + h[1, :, 0] + 1.0
        dis = lax.rsqrt(deg)
        y = jnp.dot(x_ref[...], w_ref[...], preferred_element_type=jnp.float32)
        y_ref[...] = y * dis[:, None]
        dis_ref[...] = dis[:, None]

    return pl.pallas_call(
        body,
        grid=(NP // 1024,),
        in_specs=[
            pl.BlockSpec((2, 1024, 16), lambda i: (0, i, 0)),
            pl.BlockSpec((1024, D), lambda i: (i, 0)),
            pl.BlockSpec((D, D), lambda i: (0, 0)),
        ],
        out_specs=[
            pl.BlockSpec((1024, D), lambda i: (i, 0)),
            pl.BlockSpec((1024, 1), lambda i: (i, 0)),
        ],
        out_shape=[
            jax.ShapeDtypeStruct((NP, D), jnp.float32),
            jax.ShapeDtypeStruct((NP, 1), jnp.float32),
        ],
    )(hist, x_p, W1)


def _mid_kernel(p, y1, dis, b1, W2):
    def body(p_ref, y1_ref, dis_ref, b1_ref, w_ref, y2_ref):
        ph = p_ref[...]
        agg = jnp.concatenate([ph[0], ph[1]], axis=1)
        dis = dis_ref[...]
        h1 = dis * (agg + y1_ref[...]) + b1_ref[...]
        h1 = jnp.maximum(h1, 0.0)
        y2 = jnp.dot(h1, w_ref[...], preferred_element_type=jnp.float32)
        y2_ref[...] = y2 * dis

    blk = lambda i: (i, 0)
    return pl.pallas_call(
        body,
        grid=(NP // 1024,),
        in_specs=[
            pl.BlockSpec((2, 1024, DH), lambda i: (0, i, 0)),
            pl.BlockSpec((1024, D), blk),
            pl.BlockSpec((1024, 1), blk),
            pl.BlockSpec((1, D), lambda i: (0, 0)),
            pl.BlockSpec((D, D), lambda i: (0, 0)),
        ],
        out_specs=pl.BlockSpec((1024, D), blk),
        out_shape=jax.ShapeDtypeStruct((NP, D), jnp.float32),
    )(p, y1, dis, b1, W2)


def _final_kernel(q, y2, dis, b2):
    def body(q_ref, y2_ref, dis_ref, b2_ref, h2_ref, v_ref):
        qh = q_ref[...]
        agg = jnp.concatenate([qh[0], qh[1]], axis=1)
        h2 = dis_ref[...] * (agg + y2_ref[...]) + b2_ref[...]
        h2_ref[...] = h2
        v_ref[...] = h2[:, D - 1:D]

    blk = lambda i: (i, 0)
    return pl.pallas_call(
        body,
        grid=(NP // 1024,),
        in_specs=[
            pl.BlockSpec((2, 1024, DH), lambda i: (0, i, 0)),
            pl.BlockSpec((1024, D), blk),
            pl.BlockSpec((1024, 1), blk),
            pl.BlockSpec((1, D), lambda i: (0, 0)),
        ],
        out_specs=[
            pl.BlockSpec((1024, D), blk),
            pl.BlockSpec((1024, 1), blk),
        ],
        out_shape=[
            jax.ShapeDtypeStruct((NP, D), jnp.float32),
            jax.ShapeDtypeStruct((NP, 1), jnp.float32),
        ],
    )(q, y2, dis, b2)


def _rank_kernel(vrow, brow, vcol, bcol):
    NJ = NP // 2048

    def body(vr_ref, br_ref, vc_ref, bc_ref, fo_ref, acc_ref):
        i = pl.program_id(0)
        j = pl.program_id(1)
        vi = vc_ref[...]
        bi = bc_ref[...]
        vj = vr_ref[...]
        bj = br_ref[...]
        @pl.when(j == 0)
        def _():
            acc_ref[...] = jnp.zeros((256, 1), jnp.int32)

        overlap = (bj[0, 2047] >= bi[0, 0]) & (bj[0, 0] <= bi[255, 0])

        @pl.when(overlap)
        def _():
            jg = j * 2048 + lax.broadcasted_iota(jnp.int32, (1, 2048), 1)
            ig = i * 256 + lax.broadcasted_iota(jnp.int32, (256, 1), 0)
            same = bj == bi
            before = (vj > vi) | ((vj == vi) & (jg < ig))
            cnt = jnp.sum((same & before).astype(jnp.int32), axis=1, keepdims=True)
            acc_ref[...] = acc_ref[...] + cnt

        @pl.when(j == NJ - 1)
        def _():
            rank = acc_ref[...]
            ok = (rank < K) & (bi < G)
            ig2 = i * 256 + lax.broadcasted_iota(jnp.int32, (256, 1), 0)
            trash = TRASH + ig2 // ROWS_PER_TILE
            fo_ref[...] = jnp.where(ok, bi * K + rank, trash)

    return pl.pallas_call(
        body,
        grid=(NP // 256, NJ),
        in_specs=[
            pl.BlockSpec((1, 2048), lambda i, j: (0, j)),
            pl.BlockSpec((1, 2048), lambda i, j: (0, j)),
            pl.BlockSpec((256, 1), lambda i, j: (i, 0)),
            pl.BlockSpec((256, 1), lambda i, j: (i, 0)),
        ],
        out_specs=pl.BlockSpec((256, 1), lambda i, j: (i, 0)),
        out_shape=jax.ShapeDtypeStruct((NP, 1), jnp.int32),
        scratch_shapes=[pltpu.VMEM((256, 1), jnp.int32)],
    )(vrow, brow, vcol, bcol)


def kernel(x, edge_index, batch, W1, b1, W2, b2):
    src = edge_index[0].astype(jnp.int32)
    dst = edge_index[1].astype(jnp.int32)
    si = src * 2
    sidx2 = jnp.stack([si, si + 1]).reshape(2, ER, CE)
    dst2 = dst.reshape(ER, CE)
    b32 = batch.astype(jnp.int32)
    bpad = jnp.concatenate([b32, jnp.full((NP - N,), G, jnp.int32)])
    x_p = jnp.concatenate([x, jnp.zeros((NP - N, D), x.dtype)])

    hist = _deg_kernel(dst2)
    y1, dis = _scale_matmul_kernel(hist, x_p, W1.astype(jnp.float32))

    p = _agg_kernel(y1.reshape(2 * NP, DH), sidx2, dst2)
    y2 = _mid_kernel(p, y1, dis, b1.reshape(1, D), W2.astype(jnp.float32))

    q = _agg_kernel(y2.reshape(2 * NP, DH), sidx2, dst2)
    h2, v = _final_kernel(q, y2, dis, b2.reshape(1, D))

    fo = _rank_kernel(v.reshape(1, NP), bpad.reshape(1, NP), v, bpad.reshape(NP, 1))
    out = _pool_scatter_kernel(h2, fo.reshape(NP // CP, CP))
    return out[:TRASH].reshape(G, K * D)


# speedup vs baseline: 20.5154x; 1.0029x over previous
import functools

import jax
import jax.numpy as jnp
from jax import lax
from jax.experimental import pallas as pl
from jax.experimental.pallas import tpu as pltpu
from jax.experimental.pallas import tpu_sc as plsc

N = 10000
D = 128
E = 320000
G = 64
K = 30
NP = 10240
CE = 125
ER = E // CE
RPT = ER // 32
ROWS_PER_TILE = NP // 16
ZR = 128
CP = 80
OUTROWS = 1984
TRASH = G * K

_mesh = lambda: plsc.VectorSubcoreMesh(core_axis_name="c", subcore_axis_name="s")


def _deg_kernel(dst2):
    @functools.partial(
        pl.kernel,
        mesh=_mesh(),
        compiler_params=pltpu.CompilerParams(use_tc_tiling_on_sc=False),
        out_type=jax.ShapeDtypeStruct((2, NP, 16), jnp.float32),
        scratch_types=[
            pltpu.VMEM((RPT, CE), jnp.int32),
            pltpu.VMEM((CE, 16), jnp.float32),
            pltpu.VMEM((ZR, 16), jnp.float32),
            pltpu.VMEM_SHARED((NP, 16), jnp.float32),
            pltpu.SemaphoreType.DMA,
        ],
    )
    def k(dst_hbm, out_hbm, didx, ones, zbuf, acc, sem):
        c = lax.axis_index("c")
        s = lax.axis_index("s")
        zeros16 = jnp.zeros((16,), jnp.float32)
        ones16 = jnp.ones((16,), jnp.float32)

        def fill(i, _):
            ones[i, :] = ones16
            return _
        lax.fori_loop(0, CE, fill, None)

        def zfill(i, _):
            zbuf[i, :] = zeros16
            return _
        lax.fori_loop(0, ZR, zfill, None)

        def zacc(q, _):
            pltpu.sync_copy(zbuf, acc.at[pl.ds(s * ROWS_PER_TILE + q * ZR, ZR)])
            return _
        lax.fori_loop(0, ROWS_PER_TILE // ZR, zacc, None)
        plsc.subcore_barrier()

        base = (c * 16 + s) * RPT
        pltpu.sync_copy(dst_hbm.at[pl.ds(base, RPT)], didx)

        def body(j, _):
            pltpu.async_copy(ones, acc.at[didx.at[j]], sem, add=True)
            return _
        lax.fori_loop(0, RPT, body, None)

        def drain(j, _):
            pltpu.make_async_copy(ones, acc.at[didx.at[j]], sem).wait()
            return _
        lax.fori_loop(0, RPT, drain, None)

        plsc.subcore_barrier()
        sl = pl.ds(s * ROWS_PER_TILE, ROWS_PER_TILE)
        pltpu.sync_copy(acc.at[sl], out_hbm.at[c, sl])

    return k(dst2)


DH = D // 2
RPT2 = ER // 16


def _agg_kernel(yr, sidx2, dst2):
    @functools.partial(
        pl.kernel,
        mesh=_mesh(),
        compiler_params=pltpu.CompilerParams(use_tc_tiling_on_sc=False),
        out_type=jax.ShapeDtypeStruct((2, NP, DH), jnp.float32),
        scratch_types=[
            pltpu.VMEM((RPT2, CE), jnp.int32),
            pltpu.VMEM((RPT2, CE), jnp.int32),
            pltpu.VMEM((CE, DH), jnp.float32),
            pltpu.VMEM((CE, DH), jnp.float32),
            pltpu.VMEM((CE, DH), jnp.float32),
            pltpu.VMEM((CE, DH), jnp.float32),
            pltpu.VMEM((ZR, DH), jnp.float32),
            pltpu.VMEM_SHARED((NP, DH), jnp.float32),
            pltpu.SemaphoreType.DMA,
            pltpu.SemaphoreType.DMA,
            pltpu.SemaphoreType.DMA,
            pltpu.SemaphoreType.DMA,
            pltpu.SemaphoreType.DMA,
            pltpu.SemaphoreType.DMA,
            pltpu.SemaphoreType.DMA,
            pltpu.SemaphoreType.DMA,
        ],
    )
    def k(y_hbm, src_hbm, dst_hbm, out_hbm, sidx, didx, r0, r1, r2, r3,
          zbuf, acc, g0, g1, g2, g3, s0, s1, s2, s3):
        c = lax.axis_index("c")
        s = lax.axis_index("s")
        zeros16 = jnp.zeros((16,), jnp.float32)

        def zrow(i, _):
            def zcol(j, __):
                zbuf[i, pl.ds(j * 16, 16)] = zeros16
                return __
            return lax.fori_loop(0, DH // 16, zcol, _)
        lax.fori_loop(0, ZR, zrow, None)

        def zacc(q, _):
            pltpu.sync_copy(zbuf, acc.at[pl.ds(s * ROWS_PER_TILE + q * ZR, ZR)])
            return _
        lax.fori_loop(0, ROWS_PER_TILE // ZR, zacc, None)
        plsc.subcore_barrier()

        base = s * RPT2
        pltpu.sync_copy(src_hbm.at[c, pl.ds(base, RPT2)], sidx)
        pltpu.sync_copy(dst_hbm.at[pl.ds(base, RPT2)], didx)

        bufs = (r0, r1, r2, r3)
        gsem = (g0, g1, g2, g3)
        ssem = (s0, s1, s2, s3)

        def gth(j, u):
            return pltpu.async_copy(y_hbm.at[sidx.at[j]], bufs[u], gsem[u])

        def gth_wait(j, u):
            pltpu.make_async_copy(y_hbm.at[sidx.at[j]], bufs[u], gsem[u]).wait()

        def sct(j, u):
            return pltpu.async_copy(bufs[u], acc.at[didx.at[j]], ssem[u],
                                    add=True)

        def sct_wait(j, u):
            pltpu.make_async_copy(bufs[u], acc.at[didx.at[j]], ssem[u]).wait()

        gth(0, 0)
        gth(1, 1)
        gth_wait(0, 0)
        sct(0, 0)
        gth(2, 2)
        gth_wait(1, 1)
        sct(1, 1)
        gth(3, 3)

        def body(t, _):
            for u_ in range(4):
                j = 4 * t + 2 + u_
                u = (2 + u_) % 4
                w = u_
                sct_wait(j - 2, w)
                gth(j + 2, w)
                gth_wait(j, u)
                sct(j, u)
            return _
        lax.fori_loop(0, (RPT2 - 4) // 4, body, None)

        gth_wait(RPT2 - 2, (RPT2 - 2) % 4)
        sct(RPT2 - 2, (RPT2 - 2) % 4)
        gth_wait(RPT2 - 1, (RPT2 - 1) % 4)
        sct(RPT2 - 1, (RPT2 - 1) % 4)
        for u_ in range(4):
            sct_wait(RPT2 - 4 + u_, (RPT2 - 4 + u_) % 4)

        plsc.subcore_barrier()
        sl = pl.ds(s * ROWS_PER_TILE, ROWS_PER_TILE)
        pltpu.sync_copy(acc.at[sl], out_hbm.at[c, sl])

    return k(yr, sidx2, dst2)


def _pool_scatter_kernel(h2, fo2):
    @functools.partial(
        pl.kernel,
        mesh=_mesh(),
        compiler_params=pltpu.CompilerParams(use_tc_tiling_on_sc=False),
        out_type=jax.ShapeDtypeStruct((OUTROWS, D), jnp.float32),
        scratch_types=[
            pltpu.VMEM((8, CP), jnp.int32),
            pltpu.VMEM((124, D), jnp.float32),
            pltpu.VMEM((CP, D), jnp.float32),
            pltpu.VMEM((CP, D), jnp.float32),
            pltpu.SemaphoreType.DMA,
            pltpu.SemaphoreType.DMA,
        ],
    )
    def k(h2_hbm, fo_hbm, out_hbm, fidx, zbuf, rows_a, rows_b, sem_a, sem_b):
        c = lax.axis_index("c")
        s = lax.axis_index("s")
        zeros16 = jnp.zeros((16,), jnp.float32)

        @pl.when(c == 0)
        def _():
            def zrow(i, _):
                def zcol(j, __):
                    zbuf[i, pl.ds(j * 16, 16)] = zeros16
                    return __
                return lax.fori_loop(0, D // 16, zcol, _)
            lax.fori_loop(0, 124, zrow, None)
            pltpu.sync_copy(fo_hbm.at[pl.ds(s * 8, 8)], fidx)
            pltpu.sync_copy(zbuf, out_hbm.at[pl.ds(s * 124, 124)])
            plsc.subcore_barrier()

            def src_at(q):
                return h2_hbm.at[pl.ds(s * ROWS_PER_TILE + q * CP, CP)]

            pltpu.async_copy(src_at(0), rows_a, sem_a)

            def body(t, _):
                qa = 2 * t
                qb = 2 * t + 1
                pltpu.async_copy(src_at(qb), rows_b, sem_b)
                pltpu.make_async_copy(src_at(qa), rows_a, sem_a).wait()
                pltpu.sync_copy(rows_a, out_hbm.at[fidx.at[qa]])
                qn = jnp.minimum(qa + 2, 7)
                pltpu.async_copy(src_at(qn), rows_a, sem_a)
                pltpu.make_async_copy(src_at(qb), rows_b, sem_b).wait()
                pltpu.sync_copy(rows_b, out_hbm.at[fidx.at[qb]])
                return _
            lax.fori_loop(0, 4, body, None)
            pltpu.make_async_copy(src_at(7), rows_a, sem_a).wait()

    return k(h2, fo2)


def _scale_matmul_kernel(hist, x_p, W1):
    def body(hist_ref, x_ref, w_ref, y_ref, dis_ref):
        h = hist_ref[...]
        deg = h[0, :, 0] + h[1, :, 0] + 1.0
        dis = lax.rsqrt(deg)
        y = jnp.dot(x_ref[...], w_ref[...], preferred_element_type=jnp.float32)
        y_ref[...] = y * dis[:, None]
        dis_ref[...] = dis[:, None]

    return pl.pallas_call(
        body,
        grid=(NP // 1024,),
        in_specs=[
            pl.BlockSpec((2, 1024, 16), lambda i: (0, i, 0)),
            pl.BlockSpec((1024, D), lambda i: (i, 0)),
            pl.BlockSpec((D, D), lambda i: (0, 0)),
        ],
        out_specs=[
            pl.BlockSpec((1024, D), lambda i: (i, 0)),
            pl.BlockSpec((1024, 1), lambda i: (i, 0)),
        ],
        out_shape=[
            jax.ShapeDtypeStruct((NP, D), jnp.float32),
            jax.ShapeDtypeStruct((NP, 1), jnp.float32),
        ],
    )(hist, x_p, W1)


def _mid_kernel(p, y1, dis, b1, W2):
    def body(p_ref, y1_ref, dis_ref, b1_ref, w_ref, y2_ref):
        ph = p_ref[...]
        agg = jnp.concatenate([ph[0], ph[1]], axis=1)
        dis = dis_ref[...]
        h1 = dis * (agg + y1_ref[...]) + b1_ref[...]
        h1 = jnp.maximum(h1, 0.0)
        y2 = jnp.dot(h1, w_ref[...], preferred_element_type=jnp.float32)
        y2_ref[...] = y2 * dis

    blk = lambda i: (i, 0)
    return pl.pallas_call(
        body,
        grid=(NP // 1024,),
        in_specs=[
            pl.BlockSpec((2, 1024, DH), lambda i: (0, i, 0)),
            pl.BlockSpec((1024, D), blk),
            pl.BlockSpec((1024, 1), blk),
            pl.BlockSpec((1, D), lambda i: (0, 0)),
            pl.BlockSpec((D, D), lambda i: (0, 0)),
        ],
        out_specs=pl.BlockSpec((1024, D), blk),
        out_shape=jax.ShapeDtypeStruct((NP, D), jnp.float32),
    )(p, y1, dis, b1, W2)


def _final_kernel(q, y2, dis, b2):
    def body(q_ref, y2_ref, dis_ref, b2_ref, h2_ref, v_ref):
        qh = q_ref[...]
        agg = jnp.concatenate([qh[0], qh[1]], axis=1)
        h2 = dis_ref[...] * (agg + y2_ref[...]) + b2_ref[...]
        h2_ref[...] = h2
        v_ref[...] = h2[:, D - 1:D]

    blk = lambda i: (i, 0)
    return pl.pallas_call(
        body,
        grid=(NP // 1024,),
        in_specs=[
            pl.BlockSpec((2, 1024, DH), lambda i: (0, i, 0)),
            pl.BlockSpec((1024, D), blk),
            pl.BlockSpec((1024, 1), blk),
            pl.BlockSpec((1, D), lambda i: (0, 0)),
        ],
        out_specs=[
            pl.BlockSpec((1024, D), blk),
            pl.BlockSpec((1024, 1), blk),
        ],
        out_shape=[
            jax.ShapeDtypeStruct((NP, D), jnp.float32),
            jax.ShapeDtypeStruct((NP, 1), jnp.float32),
        ],
    )(q, y2, dis, b2)


def _rank_kernel(vrow, brow, vcol, bcol):
    CW = 512
    NJ = NP // CW

    def body(vr_ref, br_ref, vc_ref, bc_ref, fo_ref):
        i = pl.program_id(0)
        vi = vc_ref[...]
        bi = bc_ref[...]
        ig = i * 256 + lax.broadcasted_iota(jnp.int32, (256, 1), 0)

        def chunk(j, acc):
            vj = vr_ref[pl.ds(j, 1), :]
            bj = br_ref[pl.ds(j, 1), :]
            overlap = (bj[0, CW - 1] >= bi[0, 0]) & (bj[0, 0] <= bi[255, 0])

            def do(a):
                jg = j * CW + lax.broadcasted_iota(jnp.int32, (1, CW), 1)
                same = bj == bi
                before = (vj > vi) | ((vj == vi) & (jg < ig))
                cnt = jnp.sum((same & before).astype(jnp.int32), axis=1,
                              keepdims=True)
                return a + cnt

            return lax.cond(overlap, do, lambda a: a, acc)

        rank = lax.fori_loop(0, NJ, chunk, jnp.zeros((256, 1), jnp.int32))
        ok = (rank < K) & (bi < G)
        trash = TRASH + ig // ROWS_PER_TILE
        fo_ref[...] = jnp.where(ok, bi * K + rank, trash)

    return pl.pallas_call(
        body,
        grid=(NP // 256,),
        in_specs=[
            pl.BlockSpec((NJ, CW), lambda i: (0, 0)),
            pl.BlockSpec((NJ, CW), lambda i: (0, 0)),
            pl.BlockSpec((256, 1), lambda i: (i, 0)),
            pl.BlockSpec((256, 1), lambda i: (i, 0)),
        ],
        out_specs=pl.BlockSpec((256, 1), lambda i: (i, 0)),
        out_shape=jax.ShapeDtypeStruct((NP, 1), jnp.int32),
    )(vrow, brow, vcol, bcol)


def kernel(x, edge_index, batch, W1, b1, W2, b2):
    src = edge_index[0].astype(jnp.int32)
    dst = edge_index[1].astype(jnp.int32)
    si = src * 2
    sidx2 = jnp.stack([si, si + 1]).reshape(2, ER, CE)
    dst2 = dst.reshape(ER, CE)
    b32 = batch.astype(jnp.int32)
    bpad = jnp.concatenate([b32, jnp.full((NP - N,), G, jnp.int32)])
    x_p = jnp.concatenate([x, jnp.zeros((NP - N, D), x.dtype)])

    hist = _deg_kernel(dst2)
    y1, dis = _scale_matmul_kernel(hist, x_p, W1.astype(jnp.float32))

    p = _agg_kernel(y1.reshape(2 * NP, DH), sidx2, dst2)
    y2 = _mid_kernel(p, y1, dis, b1.reshape(1, D), W2.astype(jnp.float32))

    q = _agg_kernel(y2.reshape(2 * NP, DH), sidx2, dst2)
    h2, v = _final_kernel(q, y2, dis, b2.reshape(1, D))

    fo = _rank_kernel(v.reshape(NP // 512, 512), bpad.reshape(NP // 512, 512),
                      v, bpad.reshape(NP, 1))
    out = _pool_scatter_kernel(h2, fo.reshape(NP // CP, CP))
    return out[:TRASH].reshape(G, K * D)


# speedup vs baseline: 22.1760x; 1.0809x over previous
import functools

import jax
import jax.numpy as jnp
from jax import lax
from jax.experimental import pallas as pl
from jax.experimental.pallas import tpu as pltpu
from jax.experimental.pallas import tpu_sc as plsc

N = 10000
D = 128
E = 320000
G = 64
K = 30
NP = 10240
CE = 125
ER = E // CE
RPT = ER // 32
ROWS_PER_TILE = NP // 16
ZR = 128
CP = 80
OUTROWS = 1984
TRASH = G * K

_mesh = lambda: plsc.VectorSubcoreMesh(core_axis_name="c", subcore_axis_name="s")


def _deg_kernel(dst2):
    @functools.partial(
        pl.kernel,
        mesh=_mesh(),
        compiler_params=pltpu.CompilerParams(use_tc_tiling_on_sc=False),
        out_type=jax.ShapeDtypeStruct((2, NP, 16), jnp.float32),
        scratch_types=[
            pltpu.VMEM((RPT, CE), jnp.int32),
            pltpu.VMEM((CE, 16), jnp.float32),
            pltpu.VMEM((ZR, 16), jnp.float32),
            pltpu.VMEM_SHARED((NP, 16), jnp.float32),
            pltpu.SemaphoreType.DMA,
        ],
    )
    def k(dst_hbm, out_hbm, didx, ones, zbuf, acc, sem):
        c = lax.axis_index("c")
        s = lax.axis_index("s")
        zeros16 = jnp.zeros((16,), jnp.float32)
        ones16 = jnp.ones((16,), jnp.float32)

        def fill(i, _):
            ones[i, :] = ones16
            return _
        lax.fori_loop(0, CE, fill, None)

        def zfill(i, _):
            zbuf[i, :] = zeros16
            return _
        lax.fori_loop(0, ZR, zfill, None)

        def zacc(q, _):
            pltpu.sync_copy(zbuf, acc.at[pl.ds(s * ROWS_PER_TILE + q * ZR, ZR)])
            return _
        lax.fori_loop(0, ROWS_PER_TILE // ZR, zacc, None)
        plsc.subcore_barrier()

        base = (c * 16 + s) * RPT
        pltpu.sync_copy(dst_hbm.at[pl.ds(base, RPT)], didx)

        def body(j, _):
            pltpu.async_copy(ones, acc.at[didx.at[j]], sem, add=True)
            return _
        lax.fori_loop(0, RPT, body, None)

        def drain(j, _):
            pltpu.make_async_copy(ones, acc.at[didx.at[j]], sem).wait()
            return _
        lax.fori_loop(0, RPT, drain, None)

        plsc.subcore_barrier()
        sl = pl.ds(s * ROWS_PER_TILE, ROWS_PER_TILE)
        pltpu.sync_copy(acc.at[sl], out_hbm.at[c, sl])

    return k(dst2)


DH = D // 2
RPT2 = ER // 16


def _agg_kernel(yr, sidx2, dst2):
    @functools.partial(
        pl.kernel,
        mesh=_mesh(),
        compiler_params=pltpu.CompilerParams(use_tc_tiling_on_sc=False),
        out_type=jax.ShapeDtypeStruct((2, NP, DH), jnp.float32),
        scratch_types=[
            pltpu.VMEM((RPT2, CE), jnp.int32),
            pltpu.VMEM((RPT2, CE), jnp.int32),
        ] + [pltpu.VMEM((CE, DH), jnp.float32)] * 6 + [
            pltpu.VMEM_SHARED((NP, DH), jnp.float32),
        ] + [pltpu.SemaphoreType.DMA] * 12,
    )
    def k(y_hbm, src_hbm, dst_hbm, out_hbm, sidx, didx,
          r0, r1, r2, r3, r4, r5, acc,
          g0, g1, g2, g3, g4, g5,
          s0, s1, s2, s3, s4, s5):
        c = lax.axis_index("c")
        s = lax.axis_index("s")
        zeros16 = jnp.zeros((16,), jnp.float32)

        def zrow(i, _):
            def zcol(j, __):
                r0[i, pl.ds(j * 16, 16)] = zeros16
                return __
            return lax.fori_loop(0, DH // 16, zcol, _)
        lax.fori_loop(0, CE, zrow, None)

        def zacc(q, _):
            pltpu.sync_copy(r0.at[pl.ds(0, 80)],
                            acc.at[pl.ds(s * ROWS_PER_TILE + q * 80, 80)])
            return _
        lax.fori_loop(0, ROWS_PER_TILE // 80, zacc, None)
        plsc.subcore_barrier()

        base = s * RPT2
        pltpu.sync_copy(src_hbm.at[c, pl.ds(base, RPT2)], sidx)
        pltpu.sync_copy(dst_hbm.at[pl.ds(base, RPT2)], didx)

        bufs = (r0, r1, r2, r3, r4, r5)
        gsem = (g0, g1, g2, g3, g4, g5)
        ssem = (s0, s1, s2, s3, s4, s5)

        def gth(j, u):
            return pltpu.async_copy(y_hbm.at[sidx.at[j]], bufs[u], gsem[u])

        def gth_wait(j, u):
            pltpu.make_async_copy(y_hbm.at[sidx.at[j]], bufs[u], gsem[u]).wait()

        def sct(j, u):
            return pltpu.async_copy(bufs[u], acc.at[didx.at[j]], ssem[u],
                                    add=True)

        def sct_wait(j, u):
            pltpu.make_async_copy(bufs[u], acc.at[didx.at[j]], ssem[u]).wait()

        for j0 in range(3):
            gth(j0, j0)
        for j0 in range(3):
            gth(j0 + 3, j0 + 3)
            gth_wait(j0, j0)
            sct(j0, j0)

        def body(t, _):
            for u_ in range(6):
                j = 6 * t + 3 + u_
                u = (3 + u_) % 6
                w = u_
                sct_wait(j - 3, w)
                gth(j + 3, w)
                gth_wait(j, u)
                sct(j, u)
            return _
        lax.fori_loop(0, (RPT2 - 10) // 6, body, None)

        for j0 in range(RPT2 - 7, RPT2):
            sct_wait(j0 - 3, (j0 - 3) % 6)
            if j0 + 3 < RPT2:
                gth(j0 + 3, (j0 + 3) % 6)
            gth_wait(j0, j0 % 6)
            sct(j0, j0 % 6)
        for j0 in range(RPT2 - 3, RPT2):
            sct_wait(j0, j0 % 6)

        plsc.subcore_barrier()
        sl = pl.ds(s * ROWS_PER_TILE, ROWS_PER_TILE)
        pltpu.sync_copy(acc.at[sl], out_hbm.at[c, sl])

    return k(yr, sidx2, dst2)


def _agg16_kernel(y16, src16, dst2):
    @functools.partial(
        pl.kernel,
        mesh=_mesh(),
        compiler_params=pltpu.CompilerParams(use_tc_tiling_on_sc=False),
        out_type=jax.ShapeDtypeStruct((2, NP, 16), jnp.float32),
        scratch_types=[
            pltpu.VMEM((RPT, CE), jnp.int32),
            pltpu.VMEM((RPT, CE), jnp.int32),
            pltpu.VMEM((CE, 16), jnp.float32),
            pltpu.VMEM((CE, 16), jnp.float32),
            pltpu.VMEM((ZR, 16), jnp.float32),
            pltpu.VMEM_SHARED((NP, 16), jnp.float32),
            pltpu.SemaphoreType.DMA,
            pltpu.SemaphoreType.DMA,
        ],
    )
    def k(y_hbm, src_hbm, dst_hbm, out_hbm, sidx, didx, ba, bb, zbuf, acc,
          sem_a, sem_b):
        c = lax.axis_index("c")
        s = lax.axis_index("s")
        zeros16 = jnp.zeros((16,), jnp.float32)

        def zfill(i, _):
            zbuf[i, :] = zeros16
            return _
        lax.fori_loop(0, ZR, zfill, None)

        def zacc(q, _):
            pltpu.sync_copy(zbuf, acc.at[pl.ds(s * ROWS_PER_TILE + q * ZR, ZR)])
            return _
        lax.fori_loop(0, ROWS_PER_TILE // ZR, zacc, None)
        plsc.subcore_barrier()

        base = (c * 16 + s) * RPT
        pltpu.sync_copy(src_hbm.at[pl.ds(base, RPT)], sidx)
        pltpu.sync_copy(dst_hbm.at[pl.ds(base, RPT)], didx)

        pltpu.async_copy(y_hbm.at[sidx.at[0]], ba, sem_a)

        def body(t, _):
            ja = 2 * t
            jb = 2 * t + 1
            pltpu.async_copy(y_hbm.at[sidx.at[jb]], bb, sem_b)
            pltpu.make_async_copy(y_hbm.at[sidx.at[ja]], ba, sem_a).wait()
            pltpu.sync_copy(ba, acc.at[didx.at[ja]], add=True)
            jn = jnp.minimum(ja + 2, RPT - 1)
            pltpu.async_copy(y_hbm.at[sidx.at[jn]], ba, sem_a)
            pltpu.make_async_copy(y_hbm.at[sidx.at[jb]], bb, sem_b).wait()
            pltpu.sync_copy(bb, acc.at[didx.at[jb]], add=True)
            return _
        lax.fori_loop(0, RPT // 2, body, None)
        pltpu.make_async_copy(y_hbm.at[sidx.at[RPT - 1]], ba, sem_a).wait()

        plsc.subcore_barrier()
        sl = pl.ds(s * ROWS_PER_TILE, ROWS_PER_TILE)
        pltpu.sync_copy(acc.at[sl], out_hbm.at[c, sl])

    return k(y16, src16, dst2)


def _pool_scatter_kernel(h2, fo2):
    @functools.partial(
        pl.kernel,
        mesh=_mesh(),
        compiler_params=pltpu.CompilerParams(use_tc_tiling_on_sc=False),
        out_type=jax.ShapeDtypeStruct((OUTROWS, D), jnp.float32),
        scratch_types=[
            pltpu.VMEM((8, CP), jnp.int32),
            pltpu.VMEM((124, D), jnp.float32),
            pltpu.VMEM((CP, D), jnp.float32),
            pltpu.VMEM((CP, D), jnp.float32),
            pltpu.SemaphoreType.DMA,
            pltpu.SemaphoreType.DMA,
        ],
    )
    def k(h2_hbm, fo_hbm, out_hbm, fidx, zbuf, rows_a, rows_b, sem_a, sem_b):
        c = lax.axis_index("c")
        s = lax.axis_index("s")
        zeros16 = jnp.zeros((16,), jnp.float32)

        @pl.when(c == 0)
        def _():
            def zrow(i, _):
                def zcol(j, __):
                    zbuf[i, pl.ds(j * 16, 16)] = zeros16
                    return __
                return lax.fori_loop(0, D // 16, zcol, _)
            lax.fori_loop(0, 124, zrow, None)
            pltpu.sync_copy(fo_hbm.at[pl.ds(s * 8, 8)], fidx)
            pltpu.sync_copy(zbuf, out_hbm.at[pl.ds(s * 124, 124)])
            plsc.subcore_barrier()

            def src_at(q):
                return h2_hbm.at[pl.ds(s * ROWS_PER_TILE + q * CP, CP)]

            pltpu.async_copy(src_at(0), rows_a, sem_a)

            def body(t, _):
                qa = 2 * t
                qb = 2 * t + 1
                pltpu.async_copy(src_at(qb), rows_b, sem_b)
                pltpu.make_async_copy(src_at(qa), rows_a, sem_a).wait()
                pltpu.sync_copy(rows_a, out_hbm.at[fidx.at[qa]])
                qn = jnp.minimum(qa + 2, 7)
                pltpu.async_copy(src_at(qn), rows_a, sem_a)
                pltpu.make_async_copy(src_at(qb), rows_b, sem_b).wait()
                pltpu.sync_copy(rows_b, out_hbm.at[fidx.at[qb]])
                return _
            lax.fori_loop(0, 4, body, None)
            pltpu.make_async_copy(src_at(7), rows_a, sem_a).wait()

    return k(h2, fo2)


def _scale_matmul_kernel(hist, x_p, W1):
    def body(hist_ref, x_ref, w_ref, y_ref, dis_ref):
        h = hist_ref[...]
        deg = h[0, :, 0] + h[1, :, 0] + 1.0
        dis = lax.rsqrt(deg)
        y = jnp.dot(x_ref[...], w_ref[...], preferred_element_type=jnp.float32)
        y_ref[...] = y * dis[:, None]
        dis_ref[...] = dis[:, None]

    return pl.pallas_call(
        body,
        grid=(NP // 1024,),
        in_specs=[
            pl.BlockSpec((2, 1024, 16), lambda i: (0, i, 0)),
            pl.BlockSpec((1024, D), lambda i: (i, 0)),
            pl.BlockSpec((D, D), lambda i: (0, 0)),
        ],
        out_specs=[
            pl.BlockSpec((1024, D), lambda i: (i, 0)),
            pl.BlockSpec((1024, 1), lambda i: (i, 0)),
        ],
        out_shape=[
            jax.ShapeDtypeStruct((NP, D), jnp.float32),
            jax.ShapeDtypeStruct((NP, 1), jnp.float32),
        ],
    )(hist, x_p, W1)


def _mid_kernel(p, y1, dis, b1, W2):
    def body(p_ref, y1_ref, dis_ref, b1_ref, w_ref, y2_ref):
        ph = p_ref[...]
        agg = jnp.concatenate([ph[0], ph[1]], axis=1)
        dis = dis_ref[...]
        h1 = dis * (agg + y1_ref[...]) + b1_ref[...]
        h1 = jnp.maximum(h1, 0.0)
        y2 = jnp.dot(h1, w_ref[...], preferred_element_type=jnp.float32)
        y2_ref[...] = y2 * dis

    blk = lambda i: (i, 0)
    return pl.pallas_call(
        body,
        grid=(NP // 1024,),
        in_specs=[
            pl.BlockSpec((2, 1024, DH), lambda i: (0, i, 0)),
            pl.BlockSpec((1024, D), blk),
            pl.BlockSpec((1024, 1), blk),
            pl.BlockSpec((1, D), lambda i: (0, 0)),
            pl.BlockSpec((D, D), lambda i: (0, 0)),
        ],
        out_specs=pl.BlockSpec((1024, D), blk),
        out_shape=jax.ShapeDtypeStruct((NP, D), jnp.float32),
    )(p, y1, dis, b1, W2)


def _final_kernel(q, y2, dis, b2):
    def body(q_ref, y2_ref, dis_ref, b2_ref, h2_ref):
        qh = q_ref[...]
        agg = jnp.concatenate([qh[0], qh[1]], axis=1)
        h2_ref[...] = dis_ref[...] * (agg + y2_ref[...]) + b2_ref[...]

    blk = lambda i: (i, 0)
    return pl.pallas_call(
        body,
        grid=(NP // 1024,),
        in_specs=[
            pl.BlockSpec((2, 1024, DH), lambda i: (0, i, 0)),
            pl.BlockSpec((1024, D), blk),
            pl.BlockSpec((1024, 1), blk),
            pl.BlockSpec((1, D), lambda i: (0, 0)),
        ],
        out_specs=pl.BlockSpec((1024, D), blk),
        out_shape=jax.ShapeDtypeStruct((NP, D), jnp.float32),
    )(q, y2, dis, b2)


def _v_kernel(q16, y2, dis, b2):
    def body(q_ref, y2_ref, dis_ref, b2_ref, v_ref):
        qh = q_ref[...]
        agg = qh[0, :, 15:16] + qh[1, :, 15:16]
        v_ref[...] = dis_ref[...] * (agg + y2_ref[...][:, D - 1:D]) \
            + b2_ref[...][:, D - 1:D]

    blk = lambda i: (i, 0)
    return pl.pallas_call(
        body,
        grid=(NP // 1024,),
        in_specs=[
            pl.BlockSpec((2, 1024, 16), lambda i: (0, i, 0)),
            pl.BlockSpec((1024, D), blk),
            pl.BlockSpec((1024, 1), blk),
            pl.BlockSpec((1, D), lambda i: (0, 0)),
        ],
        out_specs=pl.BlockSpec((1024, 1), blk),
        out_shape=jax.ShapeDtypeStruct((NP, 1), jnp.float32),
    )(q16, y2, dis, b2)


def _rank_kernel(vrow, brow, vcol, bcol):
    CW = 512
    NJ = NP // CW

    def body(vr_ref, br_ref, vc_ref, bc_ref, fo_ref):
        i = pl.program_id(0)
        vi = vc_ref[...]
        bi = bc_ref[...]
        ig = i * 256 + lax.broadcasted_iota(jnp.int32, (256, 1), 0)

        def chunk(j, acc):
            vj = vr_ref[pl.ds(j, 1), :]
            bj = br_ref[pl.ds(j, 1), :]
            overlap = (bj[0, CW - 1] >= bi[0, 0]) & (bj[0, 0] <= bi[255, 0])

            def do(a):
                jg = j * CW + lax.broadcasted_iota(jnp.int32, (1, CW), 1)
                same = bj == bi
                before = (vj > vi) | ((vj == vi) & (jg < ig))
                cnt = jnp.sum((same & before).astype(jnp.int32), axis=1,
                              keepdims=True)
                return a + cnt

            return lax.cond(overlap, do, lambda a: a, acc)

        rank = lax.fori_loop(0, NJ, chunk, jnp.zeros((256, 1), jnp.int32))
        ok = (rank < K) & (bi < G)
        trash = TRASH + ig // ROWS_PER_TILE
        fo_ref[...] = jnp.where(ok, bi * K + rank, trash)

    return pl.pallas_call(
        body,
        grid=(NP // 256,),
        in_specs=[
            pl.BlockSpec((NJ, CW), lambda i: (0, 0)),
            pl.BlockSpec((NJ, CW), lambda i: (0, 0)),
            pl.BlockSpec((256, 1), lambda i: (i, 0)),
            pl.BlockSpec((256, 1), lambda i: (i, 0)),
        ],
        out_specs=pl.BlockSpec((256, 1), lambda i: (i, 0)),
        out_shape=jax.ShapeDtypeStruct((NP, 1), jnp.int32),
    )(vrow, brow, vcol, bcol)


def kernel(x, edge_index, batch, W1, b1, W2, b2):
    src = edge_index[0].astype(jnp.int32)
    dst = edge_index[1].astype(jnp.int32)
    si = src * 2
    sidx2 = jnp.stack([si, si + 1]).reshape(2, ER, CE)
    src16 = (src * 8 + 7).reshape(ER, CE)
    dst2 = dst.reshape(ER, CE)
    b32 = batch.astype(jnp.int32)
    bpad = jnp.concatenate([b32, jnp.full((NP - N,), G, jnp.int32)])
    x_p = jnp.concatenate([x, jnp.zeros((NP - N, D), x.dtype)])

    hist = _deg_kernel(dst2)
    y1, dis = _scale_matmul_kernel(hist, x_p, W1.astype(jnp.float32))

    p = _agg_kernel(y1.reshape(2 * NP, DH), sidx2, dst2)
    y2 = _mid_kernel(p, y1, dis, b1.reshape(1, D), W2.astype(jnp.float32))

    q16 = _agg16_kernel(y2.reshape(NP * 8, 16), src16, dst2)
    v = _v_kernel(q16, y2, dis, b2.reshape(1, D))
    q = _agg_kernel(y2.reshape(2 * NP, DH), sidx2, dst2)
    fo = _rank_kernel(v.reshape(NP // 512, 512), bpad.reshape(NP // 512, 512),
                      v, bpad.reshape(NP, 1))
    h2 = _final_kernel(q, y2, dis, b2.reshape(1, D))
    out = _pool_scatter_kernel(h2, fo.reshape(NP // CP, CP))
    return out[:TRASH].reshape(G, K * D)


# speedup vs baseline: 23.0383x; 1.0389x over previous
import functools

import jax
import jax.numpy as jnp
from jax import lax
from jax.experimental import pallas as pl
from jax.experimental.pallas import tpu as pltpu
from jax.experimental.pallas import tpu_sc as plsc

N = 10000
D = 128
E = 320000
G = 64
K = 30
NP = 10240
CE = 125
ER = E // CE
RPT = ER // 32
ROWS_PER_TILE = NP // 16
ZR = 128
CP = 80
OUTROWS = 1984
TRASH = G * K

_mesh = lambda: plsc.VectorSubcoreMesh(core_axis_name="c", subcore_axis_name="s")


def _deg_kernel(dst2):
    @functools.partial(
        pl.kernel,
        mesh=_mesh(),
        compiler_params=pltpu.CompilerParams(use_tc_tiling_on_sc=False),
        out_type=jax.ShapeDtypeStruct((2, NP, 16), jnp.float32),
        scratch_types=[
            pltpu.VMEM((RPT, CE), jnp.int32),
            pltpu.VMEM((CE, 16), jnp.float32),
            pltpu.VMEM((ZR, 16), jnp.float32),
            pltpu.VMEM_SHARED((NP, 16), jnp.float32),
            pltpu.SemaphoreType.DMA,
        ],
    )
    def k(dst_hbm, out_hbm, didx, ones, zbuf, acc, sem):
        c = lax.axis_index("c")
        s = lax.axis_index("s")
        zeros16 = jnp.zeros((16,), jnp.float32)
        ones16 = jnp.ones((16,), jnp.float32)

        def fill(i, _):
            ones[i, :] = ones16
            return _
        lax.fori_loop(0, CE, fill, None)

        def zfill(i, _):
            zbuf[i, :] = zeros16
            return _
        lax.fori_loop(0, ZR, zfill, None)

        def zacc(q, _):
            pltpu.sync_copy(zbuf, acc.at[pl.ds(s * ROWS_PER_TILE + q * ZR, ZR)])
            return _
        lax.fori_loop(0, ROWS_PER_TILE // ZR, zacc, None)
        plsc.subcore_barrier()

        base = (c * 16 + s) * RPT
        pltpu.sync_copy(dst_hbm.at[pl.ds(base, RPT)], didx)

        def body(j, _):
            pltpu.async_copy(ones, acc.at[didx.at[j]], sem, add=True)
            return _
        lax.fori_loop(0, RPT, body, None)

        def drain(j, _):
            pltpu.make_async_copy(ones, acc.at[didx.at[j]], sem).wait()
            return _
        lax.fori_loop(0, RPT, drain, None)

        plsc.subcore_barrier()
        sl = pl.ds(s * ROWS_PER_TILE, ROWS_PER_TILE)
        pltpu.sync_copy(acc.at[sl], out_hbm.at[c, sl])

    return k(dst2)


DH = D // 2
RPT2 = ER // 16


def _agg_kernel(yr, sidx2, dst2):
    @functools.partial(
        pl.kernel,
        mesh=_mesh(),
        compiler_params=pltpu.CompilerParams(use_tc_tiling_on_sc=False),
        out_type=jax.ShapeDtypeStruct((2, NP, DH), jnp.float32),
        scratch_types=[
            pltpu.VMEM((RPT2, CE), jnp.int32),
            pltpu.VMEM((RPT2, CE), jnp.int32),
        ] + [pltpu.VMEM((CE, DH), jnp.float32)] * 6 + [
            pltpu.VMEM_SHARED((NP, DH), jnp.float32),
        ] + [pltpu.SemaphoreType.DMA] * 12,
    )
    def k(y_hbm, src_hbm, dst_hbm, out_hbm, sidx, didx,
          r0, r1, r2, r3, r4, r5, acc,
          g0, g1, g2, g3, g4, g5,
          s0, s1, s2, s3, s4, s5):
        c = lax.axis_index("c")
        s = lax.axis_index("s")
        zeros16 = jnp.zeros((16,), jnp.float32)

        def zrow(i, _):
            def zcol(j, __):
                r0[i, pl.ds(j * 16, 16)] = zeros16
                return __
            return lax.fori_loop(0, DH // 16, zcol, _)
        lax.fori_loop(0, CE, zrow, None)

        def zacc(q, _):
            pltpu.sync_copy(r0.at[pl.ds(0, 80)],
                            acc.at[pl.ds(s * ROWS_PER_TILE + q * 80, 80)])
            return _
        lax.fori_loop(0, ROWS_PER_TILE // 80, zacc, None)
        plsc.subcore_barrier()

        base = s * RPT2
        pltpu.sync_copy(src_hbm.at[c, pl.ds(base, RPT2)], sidx)
        pltpu.sync_copy(dst_hbm.at[pl.ds(base, RPT2)], didx)

        bufs = (r0, r1, r2, r3, r4, r5)
        gsem = (g0, g1, g2, g3, g4, g5)
        ssem = (s0, s1, s2, s3, s4, s5)

        def gth(j, u):
            return pltpu.async_copy(y_hbm.at[sidx.at[j]], bufs[u], gsem[u])

        def gth_wait(j, u):
            pltpu.make_async_copy(y_hbm.at[sidx.at[j]], bufs[u], gsem[u]).wait()

        def sct(j, u):
            return pltpu.async_copy(bufs[u], acc.at[didx.at[j]], ssem[u],
                                    add=True)

        def sct_wait(j, u):
            pltpu.make_async_copy(bufs[u], acc.at[didx.at[j]], ssem[u]).wait()

        for j0 in range(3):
            gth(j0, j0)
        for j0 in range(3):
            gth(j0 + 3, j0 + 3)
            gth_wait(j0, j0)
            sct(j0, j0)

        def body(t, _):
            for u_ in range(6):
                j = 6 * t + 3 + u_
                u = (3 + u_) % 6
                w = u_
                sct_wait(j - 3, w)
                gth(j + 3, w)
                gth_wait(j, u)
                sct(j, u)
            return _
        lax.fori_loop(0, (RPT2 - 10) // 6, body, None)

        for j0 in range(RPT2 - 7, RPT2):
            sct_wait(j0 - 3, (j0 - 3) % 6)
            if j0 + 3 < RPT2:
                gth(j0 + 3, (j0 + 3) % 6)
            gth_wait(j0, j0 % 6)
            sct(j0, j0 % 6)
        for j0 in range(RPT2 - 3, RPT2):
            sct_wait(j0, j0 % 6)

        plsc.subcore_barrier()
        sl = pl.ds(s * ROWS_PER_TILE, ROWS_PER_TILE)
        pltpu.sync_copy(acc.at[sl], out_hbm.at[c, sl])

    return k(yr, sidx2, dst2)


def _agg16_kernel(y16, src16, dst2):
    @functools.partial(
        pl.kernel,
        mesh=_mesh(),
        compiler_params=pltpu.CompilerParams(use_tc_tiling_on_sc=False),
        out_type=jax.ShapeDtypeStruct((2, NP, 16), jnp.float32),
        scratch_types=[
            pltpu.VMEM((RPT, CE), jnp.int32),
            pltpu.VMEM((RPT, CE), jnp.int32),
        ] + [pltpu.VMEM((CE, 16), jnp.float32)] * 6 + [
            pltpu.VMEM((ZR, 16), jnp.float32),
            pltpu.VMEM_SHARED((NP, 16), jnp.float32),
        ] + [pltpu.SemaphoreType.DMA] * 12,
    )
    def k(y_hbm, src_hbm, dst_hbm, out_hbm, sidx, didx,
          r0, r1, r2, r3, r4, r5, zbuf, acc,
          g0, g1, g2, g3, g4, g5, s0, s1, s2, s3, s4, s5):
        c = lax.axis_index("c")
        s = lax.axis_index("s")
        zeros16 = jnp.zeros((16,), jnp.float32)

        def zfill(i, _):
            zbuf[i, :] = zeros16
            return _
        lax.fori_loop(0, ZR, zfill, None)

        def zacc(q, _):
            pltpu.sync_copy(zbuf, acc.at[pl.ds(s * ROWS_PER_TILE + q * ZR, ZR)])
            return _
        lax.fori_loop(0, ROWS_PER_TILE // ZR, zacc, None)
        plsc.subcore_barrier()

        base = (c * 16 + s) * RPT
        pltpu.sync_copy(src_hbm.at[pl.ds(base, RPT)], sidx)
        pltpu.sync_copy(dst_hbm.at[pl.ds(base, RPT)], didx)

        bufs = (r0, r1, r2, r3, r4, r5)
        gsem = (g0, g1, g2, g3, g4, g5)
        ssem = (s0, s1, s2, s3, s4, s5)

        def gth(j, u):
            return pltpu.async_copy(y_hbm.at[sidx.at[j]], bufs[u], gsem[u])

        def gth_wait(j, u):
            pltpu.make_async_copy(y_hbm.at[sidx.at[j]], bufs[u], gsem[u]).wait()

        def sct(j, u):
            return pltpu.async_copy(bufs[u], acc.at[didx.at[j]], ssem[u],
                                    add=True)

        def sct_wait(j, u):
            pltpu.make_async_copy(bufs[u], acc.at[didx.at[j]], ssem[u]).wait()

        for j0 in range(3):
            gth(j0, j0)
        for j0 in range(3):
            gth(j0 + 3, j0 + 3)
            gth_wait(j0, j0)
            sct(j0, j0)

        def body(t, _):
            for u_ in range(6):
                j = 6 * t + 3 + u_
                u = (3 + u_) % 6
                w = u_
                sct_wait(j - 3, w)
                gth(j + 3, w)
                gth_wait(j, u)
                sct(j, u)
            return _
        lax.fori_loop(0, (RPT - 10) // 6, body, None)

        for j0 in range(3 + 6 * ((RPT - 10) // 6), RPT):
            sct_wait(j0 - 3, (j0 - 3) % 6)
            if j0 + 3 < RPT:
                gth(j0 + 3, (j0 + 3) % 6)
            gth_wait(j0, j0 % 6)
            sct(j0, j0 % 6)
        for j0 in range(RPT - 3, RPT):
            sct_wait(j0, j0 % 6)

        plsc.subcore_barrier()
        sl = pl.ds(s * ROWS_PER_TILE, ROWS_PER_TILE)
        pltpu.sync_copy(acc.at[sl], out_hbm.at[c, sl])

    return k(y16, src16, dst2)


def _pool_scatter_kernel(h2, fo2):
    @functools.partial(
        pl.kernel,
        mesh=_mesh(),
        compiler_params=pltpu.CompilerParams(use_tc_tiling_on_sc=False),
        out_type=jax.ShapeDtypeStruct((OUTROWS, D), jnp.float32),
        scratch_types=[
            pltpu.VMEM((8, CP), jnp.int32),
            pltpu.VMEM((124, D), jnp.float32),
            pltpu.VMEM((CP, D), jnp.float32),
            pltpu.VMEM((CP, D), jnp.float32),
            pltpu.SemaphoreType.DMA,
            pltpu.SemaphoreType.DMA,
        ],
    )
    def k(h2_hbm, fo_hbm, out_hbm, fidx, zbuf, rows_a, rows_b, sem_a, sem_b):
        c = lax.axis_index("c")
        s = lax.axis_index("s")
        zeros16 = jnp.zeros((16,), jnp.float32)

        @pl.when(c == 0)
        def _():
            def zrow(i, _):
                def zcol(j, __):
                    zbuf[i, pl.ds(j * 16, 16)] = zeros16
                    return __
                return lax.fori_loop(0, D // 16, zcol, _)
            lax.fori_loop(0, 124, zrow, None)
            pltpu.sync_copy(fo_hbm.at[pl.ds(s * 8, 8)], fidx)
            pltpu.sync_copy(zbuf, out_hbm.at[pl.ds(s * 124, 124)])
            plsc.subcore_barrier()

            def src_at(q):
                return h2_hbm.at[pl.ds(s * ROWS_PER_TILE + q * CP, CP)]

            pltpu.async_copy(src_at(0), rows_a, sem_a)

            def body(t, _):
                qa = 2 * t
                qb = 2 * t + 1
                pltpu.async_copy(src_at(qb), rows_b, sem_b)
                pltpu.make_async_copy(src_at(qa), rows_a, sem_a).wait()
                pltpu.sync_copy(rows_a, out_hbm.at[fidx.at[qa]])
                qn = jnp.minimum(qa + 2, 7)
                pltpu.async_copy(src_at(qn), rows_a, sem_a)
                pltpu.make_async_copy(src_at(qb), rows_b, sem_b).wait()
                pltpu.sync_copy(rows_b, out_hbm.at[fidx.at[qb]])
                return _
            lax.fori_loop(0, 4, body, None)
            pltpu.make_async_copy(src_at(7), rows_a, sem_a).wait()

    return k(h2, fo2)


def _scale_matmul_kernel(hist, x_p, W1):
    def body(hist_ref, x_ref, w_ref, y_ref, dis_ref):
        h = hist_ref[...]
        deg = h[0, :, 0] + h[1, :, 0] + 1.0
        dis = lax.rsqrt(deg)
        y = jnp.dot(x_ref[...], w_ref[...], preferred_element_type=jnp.float32)
        y_ref[...] = y * dis[:, None]
        dis_ref[...] = dis[:, None]

    return pl.pallas_call(
        body,
        grid=(NP // 1024,),
        in_specs=[
            pl.BlockSpec((2, 1024, 16), lambda i: (0, i, 0)),
            pl.BlockSpec((1024, D), lambda i: (i, 0)),
            pl.BlockSpec((D, D), lambda i: (0, 0)),
        ],
        out_specs=[
            pl.BlockSpec((1024, D), lambda i: (i, 0)),
            pl.BlockSpec((1024, 1), lambda i: (i, 0)),
        ],
        out_shape=[
            jax.ShapeDtypeStruct((NP, D), jnp.float32),
            jax.ShapeDtypeStruct((NP, 1), jnp.float32),
        ],
    )(hist, x_p, W1)


def _mid_kernel(p, y1, dis, b1, W2):
    def body(p_ref, y1_ref, dis_ref, b1_ref, w_ref, y2_ref, yc_ref):
        ph = p_ref[...]
        agg = jnp.concatenate([ph[0], ph[1]], axis=1)
        dis = dis_ref[...]
        h1 = dis * (agg + y1_ref[...]) + b1_ref[...]
        h1 = jnp.maximum(h1, 0.0)
        y2 = jnp.dot(h1, w_ref[...], preferred_element_type=jnp.float32) * dis
        y2_ref[...] = y2
        yc_ref[...] = y2[:, D - 1:D]

    blk = lambda i: (i, 0)
    return pl.pallas_call(
        body,
        grid=(NP // 1024,),
        in_specs=[
            pl.BlockSpec((2, 1024, DH), lambda i: (0, i, 0)),
            pl.BlockSpec((1024, D), blk),
            pl.BlockSpec((1024, 1), blk),
            pl.BlockSpec((1, D), lambda i: (0, 0)),
            pl.BlockSpec((D, D), lambda i: (0, 0)),
        ],
        out_specs=[
            pl.BlockSpec((1024, D), blk),
            pl.BlockSpec((1024, 1), blk),
        ],
        out_shape=[
            jax.ShapeDtypeStruct((NP, D), jnp.float32),
            jax.ShapeDtypeStruct((NP, 1), jnp.float32),
        ],
    )(p, y1, dis, b1, W2)


def _final_kernel(q, y2, dis, b2):
    def body(q_ref, y2_ref, dis_ref, b2_ref, h2_ref):
        qh = q_ref[...]
        agg = jnp.concatenate([qh[0], qh[1]], axis=1)
        h2_ref[...] = dis_ref[...] * (agg + y2_ref[...]) + b2_ref[...]

    blk = lambda i: (i, 0)
    return pl.pallas_call(
        body,
        grid=(NP // 1024,),
        in_specs=[
            pl.BlockSpec((2, 1024, DH), lambda i: (0, i, 0)),
            pl.BlockSpec((1024, D), blk),
            pl.BlockSpec((1024, 1), blk),
            pl.BlockSpec((1, D), lambda i: (0, 0)),
        ],
        out_specs=pl.BlockSpec((1024, D), blk),
        out_shape=jax.ShapeDtypeStruct((NP, D), jnp.float32),
    )(q, y2, dis, b2)


def _v_kernel(q16, y2col, dis, b2):
    def body(q_ref, yc_ref, dis_ref, b2_ref, v_ref):
        qh = q_ref[...]
        agg = qh[0, :, 15:16] + qh[1, :, 15:16]
        v_ref[...] = dis_ref[...] * (agg + yc_ref[...]) \
            + b2_ref[...][:, D - 1:D]

    blk = lambda i: (i, 0)
    return pl.pallas_call(
        body,
        grid=(NP // 1024,),
        in_specs=[
            pl.BlockSpec((2, 1024, 16), lambda i: (0, i, 0)),
            pl.BlockSpec((1024, 1), blk),
            pl.BlockSpec((1024, 1), blk),
            pl.BlockSpec((1, D), lambda i: (0, 0)),
        ],
        out_specs=pl.BlockSpec((1024, 1), blk),
        out_shape=jax.ShapeDtypeStruct((NP, 1), jnp.float32),
    )(q16, y2col, dis, b2)


def _rank_kernel(vrow, brow, vcol, bcol):
    CW = 512
    NJ = NP // CW

    def body(vr_ref, br_ref, vc_ref, bc_ref, fo_ref):
        i = pl.program_id(0)
        vi = vc_ref[...]
        bi = bc_ref[...]
        ig = i * 256 + lax.broadcasted_iota(jnp.int32, (256, 1), 0)

        def chunk(j, acc):
            vj = vr_ref[pl.ds(j, 1), :]
            bj = br_ref[pl.ds(j, 1), :]
            overlap = (bj[0, CW - 1] >= bi[0, 0]) & (bj[0, 0] <= bi[255, 0])

            def do(a):
                jg = j * CW + lax.broadcasted_iota(jnp.int32, (1, CW), 1)
                same = bj == bi
                before = (vj > vi) | ((vj == vi) & (jg < ig))
                cnt = jnp.sum((same & before).astype(jnp.int32), axis=1,
                              keepdims=True)
                return a + cnt

            return lax.cond(overlap, do, lambda a: a, acc)

        rank = lax.fori_loop(0, NJ, chunk, jnp.zeros((256, 1), jnp.int32))
        ok = (rank < K) & (bi < G)
        trash = TRASH + ig // ROWS_PER_TILE
        fo_ref[...] = jnp.where(ok, bi * K + rank, trash)

    return pl.pallas_call(
        body,
        grid=(NP // 256,),
        in_specs=[
            pl.BlockSpec((NJ, CW), lambda i: (0, 0)),
            pl.BlockSpec((NJ, CW), lambda i: (0, 0)),
            pl.BlockSpec((256, 1), lambda i: (i, 0)),
            pl.BlockSpec((256, 1), lambda i: (i, 0)),
        ],
        out_specs=pl.BlockSpec((256, 1), lambda i: (i, 0)),
        out_shape=jax.ShapeDtypeStruct((NP, 1), jnp.int32),
    )(vrow, brow, vcol, bcol)


def kernel(x, edge_index, batch, W1, b1, W2, b2):
    src = edge_index[0].astype(jnp.int32)
    dst = edge_index[1].astype(jnp.int32)
    si = src * 2
    sidx2 = jnp.stack([si, si + 1]).reshape(2, ER, CE)
    src16 = (src * 8 + 7).reshape(ER, CE)
    dst2 = dst.reshape(ER, CE)
    b32 = batch.astype(jnp.int32)
    bpad = jnp.concatenate([b32, jnp.full((NP - N,), G, jnp.int32)])
    x_p = jnp.concatenate([x, jnp.zeros((NP - N, D), x.dtype)])

    hist = _deg_kernel(dst2)
    y1, dis = _scale_matmul_kernel(hist, x_p, W1.astype(jnp.float32))

    p = _agg_kernel(y1.reshape(2 * NP, DH), sidx2, dst2)
    y2, y2col = _mid_kernel(p, y1, dis, b1.reshape(1, D),
                            W2.astype(jnp.float32))

    q16 = _agg16_kernel(y2.reshape(NP * 8, 16), src16, dst2)
    v = _v_kernel(q16, y2col, dis, b2.reshape(1, D))
    q = _agg_kernel(y2.reshape(2 * NP, DH), sidx2, dst2)
    fo = _rank_kernel(v.reshape(NP // 512, 512), bpad.reshape(NP // 512, 512),
                      v, bpad.reshape(NP, 1))
    h2 = _final_kernel(q, y2, dis, b2.reshape(1, D))
    out = _pool_scatter_kernel(h2, fo.reshape(NP // CP, CP))
    return out[:TRASH].reshape(G, K * D)
